# Initial kernel scaffold; baseline (speedup 1.0000x reference)
import jax
import jax.numpy as jnp
from jax.experimental import pallas as pl


def kernel(x, edge_index, W1, b1, W2, b2):
    raise NotImplementedError("write your pallas kernel here")


# speedup vs baseline: 7.1469x; 7.1469x over previous
import dataclasses
import functools

import jax
import jax.numpy as jnp
from jax import lax
from jax.experimental import pallas as pl
from jax.experimental.pallas import tpu as pltpu
from jax.experimental.pallas import tpu_sc as plsc

N = 10000
E = 160000
D = 256

NC = 2
NS = 16
L = 16
NB = 4
BS = 2504
ACC = 2512
DUMP = BS
EB = E // NS
B = 128
CAP = 10240
ZS = 152

_mesh = plsc.VectorSubcoreMesh(core_axis_name="c", subcore_axis_name="s")

_cp = pltpu.CompilerParams()
if "needs_layout_passes" in pltpu.CompilerParams.__dataclass_fields__:
    _cp = dataclasses.replace(_cp, needs_layout_passes=False)


@functools.partial(
    pl.kernel,
    out_type=[
        jax.ShapeDtypeStruct((NC * NS * 2 * CAP,), jnp.int32),
        jax.ShapeDtypeStruct((NC * NS * 2 * CAP,), jnp.int32),
        jax.ShapeDtypeStruct((NC * NS * 2 * B,), jnp.int32),
        jax.ShapeDtypeStruct((N, L), jnp.float32),
    ],
    mesh=_mesh,
    scratch_types=[
        pltpu.VMEM((EB,), jnp.int32),
        pltpu.VMEM((EB,), jnp.int32),
        pltpu.VMEM((CAP + L,), jnp.int32),
        pltpu.VMEM((CAP + L,), jnp.int32),
        pltpu.VMEM((CAP + L,), jnp.int32),
        pltpu.VMEM((CAP + L,), jnp.int32),
        pltpu.VMEM((B,), jnp.int32),
        pltpu.VMEM((B, L), jnp.float32),
        pltpu.VMEM((2, B), jnp.int32),
        pltpu.VMEM((ZS, L), jnp.float32),
        pltpu.VMEM_SHARED((ACC, L), jnp.float32),
        pltpu.SemaphoreType.DMA,
    ],
    compiler_params=_cp,
)
def _route(src_hbm, dst_hbm, seg_src, seg_dst, cnt_hbm, deg_hbm,
           src_v, dst_v, cs0_v, cd0_v, cs1_v, cd1_v, pc_v, ones_v, ix_v,
           zero_v, deg_acc, sem):
    c = lax.axis_index("c")
    s = lax.axis_index("s")
    w = c * NS + s

    @pl.loop(0, B)
    def _(i):
        ones_v[i, :] = jnp.ones((L,), jnp.float32)

    @pl.loop(0, ZS)
    def _(i):
        zero_v[i, :] = jnp.zeros((L,), jnp.float32)

    pltpu.sync_copy(src_hbm.at[pl.ds(s * EB, EB)], src_v)
    pltpu.sync_copy(dst_hbm.at[pl.ds(s * EB, EB)], dst_v)

    lane = lax.iota(jnp.int32, L)
    lo0 = (2 * c) * BS
    lo1 = (2 * c + 1) * BS

    def body(i, carry):
        cnt0, cnt1 = carry
        dv = dst_v[pl.ds(i * L, L)]
        sv = src_v[pl.ds(i * L, L)]
        m0 = (dv >= lo0) & (dv < lo0 + BS)
        m1 = (dv >= lo1) & (dv < lo1 + BS)
        sel0 = plsc.cumsum(m0.astype(jnp.int32))
        sel1 = plsc.cumsum(m1.astype(jnp.int32))
        idx0 = jnp.where(m0, cnt0 + sel0 - 1, CAP + lane)
        idx1 = jnp.where(m1, cnt1 + sel1 - 1, CAP + lane)
        plsc.store_scatter(cs0_v, [idx0], sv)
        plsc.store_scatter(cd0_v, [idx0], dv - lo0)
        plsc.store_scatter(cs1_v, [idx1], sv)
        plsc.store_scatter(cd1_v, [idx1], dv - lo1)
        return cnt0 + sel0[L - 1], cnt1 + sel1[L - 1]

    cnt0, cnt1 = lax.fori_loop(0, EB // L, body,
                               (jnp.int32(0), jnp.int32(0)), unroll=False)

    def mk_pad(cs_v, cd_v, cnt, cnt_pad):
        def pad(i, _):
            @pl.when(cnt + i * L < cnt_pad)
            def _():
                cs_v[pl.ds(cnt + i * L, L)] = jnp.zeros((L,), jnp.int32)
                cd_v[pl.ds(cnt + i * L, L)] = jnp.full((L,), DUMP, jnp.int32)
            return 0
        lax.fori_loop(0, B // L, pad, 0, unroll=False)

    cnt_pad0 = ((cnt0 + B - 1) // B) * B
    cnt_pad1 = ((cnt1 + B - 1) // B) * B
    mk_pad(cs0_v, cd0_v, cnt0, cnt_pad0)
    mk_pad(cs1_v, cd1_v, cnt1, cnt_pad1)

    pltpu.sync_copy(cs0_v.at[pl.ds(0, CAP)],
                    seg_src.at[pl.ds((w * 2) * CAP, CAP)])
    pltpu.sync_copy(cd0_v.at[pl.ds(0, CAP)],
                    seg_dst.at[pl.ds((w * 2) * CAP, CAP)])
    pltpu.sync_copy(cs1_v.at[pl.ds(0, CAP)],
                    seg_src.at[pl.ds((w * 2 + 1) * CAP, CAP)])
    pltpu.sync_copy(cd1_v.at[pl.ds(0, CAP)],
                    seg_dst.at[pl.ds((w * 2 + 1) * CAP, CAP)])

    @pl.loop(0, B // L)
    def _(i):
        pc_v[pl.ds(i * L, L)] = jnp.full((L,), cnt_pad0, jnp.int32)

    pltpu.sync_copy(pc_v, cnt_hbm.at[pl.ds((w * 2) * B, B)])

    @pl.loop(0, B // L)
    def _(i):
        pc_v[pl.ds(i * L, L)] = jnp.full((L,), cnt_pad1, jnp.int32)

    pltpu.sync_copy(pc_v, cnt_hbm.at[pl.ds((w * 2 + 1) * B, B)])

    for p, cnt_pad in ((0, cnt_pad0), (1, cnt_pad1)):
        r = 2 * c + p
        pltpu.sync_copy(zero_v, deg_acc.at[pl.ds(s * ZS, ZS)])

        @pl.when(s == 0)
        def _():
            pltpu.sync_copy(zero_v.at[pl.ds(0, ACC - NS * ZS)],
                            deg_acc.at[pl.ds(NS * ZS, ACC - NS * ZS)])

        plsc.subcore_barrier()
        nblk = cnt_pad // B
        base = (w * 2 + p) * CAP

        def hist(j, _):
            pltpu.sync_copy(seg_dst.at[pl.ds(base + j * B, B)], ix_v.at[0])
            pltpu.sync_copy(ones_v, deg_acc.at[ix_v.at[0]], add=True)
            return 0

        lax.fori_loop(0, nblk, hist, 0, unroll=False)
        plsc.subcore_barrier()

        pltpu.sync_copy(deg_acc.at[pl.ds(s * ZS, ZS)],
                        deg_hbm.at[pl.ds(r * BS + s * ZS, ZS)])

        @pl.when(s == 0)
        def _():
            pltpu.sync_copy(deg_acc.at[pl.ds(NS * ZS, 56)],
                            deg_hbm.at[pl.ds(r * BS + NS * ZS, 56)])

        @pl.when((s == 1) & (r < NB - 1))
        def _():
            pltpu.sync_copy(deg_acc.at[pl.ds(NS * ZS + 56, 16)],
                            deg_hbm.at[pl.ds(r * BS + NS * ZS + 56, 16)])

        plsc.subcore_barrier()


@functools.partial(
    pl.kernel,
    out_type=jax.ShapeDtypeStruct((N * 2, 128), jnp.float32),
    mesh=_mesh,
    scratch_types=[
        pltpu.VMEM((B,), jnp.int32),
        pltpu.VMEM((2, B), jnp.int32),
        pltpu.VMEM((2, B), jnp.int32),
        pltpu.VMEM((2, B), jnp.int32),
        pltpu.VMEM((2, B), jnp.int32),
        pltpu.VMEM((2, B, 128), jnp.float32),
        pltpu.VMEM((64, 128), jnp.float32),
        pltpu.VMEM_SHARED((ACC * 2, 128), jnp.float32),
        pltpu.SemaphoreType.DMA,
        pltpu.SemaphoreType.DMA,
    ],
    compiler_params=_cp,
)
def _edge_stage(g_hbm, seg_src, seg_dst, cnt_hbm, out_hbm,
                cv, six, dix, ixg, ixs, rows, zb, acc, sem0, sem1):
    c = lax.axis_index("c")
    s = lax.axis_index("s")
    w = c * NS + s

    @pl.loop(0, 64)
    def _(i):
        @pl.loop(0, 128 // L)
        def _(k):
            zb[i, pl.ds(k * L, L)] = jnp.zeros((L,), jnp.float32)

    for p in (0, 1):
        r = 2 * c + p
        pltpu.sync_copy(cnt_hbm.at[pl.ds((w * 2 + p) * B, B)], cv)
        nblk = cv[pl.ds(0, L)][0] // B

        a0 = s * ZS * 2

        @pl.loop(0, 4)
        def _(k):
            pltpu.sync_copy(zb, acc.at[pl.ds(a0 + k * 64, 64)])

        pltpu.sync_copy(zb.at[pl.ds(0, 2 * ZS - 256)],
                        acc.at[pl.ds(a0 + 256, 2 * ZS - 256)])

        @pl.when(s == 0)
        def _():
            t0 = NS * ZS * 2
            pltpu.sync_copy(zb, acc.at[pl.ds(t0, 64)])
            pltpu.sync_copy(zb, acc.at[pl.ds(t0 + 64, 64)])
            pltpu.sync_copy(zb.at[pl.ds(0, 2 * (ACC - NS * ZS) - 128)],
                            acc.at[pl.ds(t0 + 128,
                                         2 * (ACC - NS * ZS) - 128)])

        plsc.subcore_barrier()

        base = (w * 2 + p) * CAP

        def eb(j, _):
            pltpu.sync_copy(seg_src.at[pl.ds(base + j * B, B)], six.at[0])
            pltpu.sync_copy(seg_dst.at[pl.ds(base + j * B, B)], dix.at[0])

            @pl.loop(0, B // L)
            def _(v):
                sv = six[0, pl.ds(v * L, L)] * 2
                dv = dix[0, pl.ds(v * L, L)] * 2
                ixg[0, pl.ds(v * L, L)] = sv
                ixg[1, pl.ds(v * L, L)] = sv + 1
                ixs[0, pl.ds(v * L, L)] = dv
                ixs[1, pl.ds(v * L, L)] = dv + 1

            cp0 = pltpu.async_copy(g_hbm.at[ixg.at[0]], rows.at[0], sem0)
            cp1 = pltpu.async_copy(g_hbm.at[ixg.at[1]], rows.at[1], sem1)
            cp0.wait()
            cp1.wait()
            pltpu.sync_copy(rows.at[0], acc.at[ixs.at[0]], add=True)
            pltpu.sync_copy(rows.at[1], acc.at[ixs.at[1]], add=True)
            return 0

        lax.fori_loop(0, nblk, eb, 0, unroll=False)
        plsc.subcore_barrier()

        pltpu.sync_copy(acc.at[pl.ds(a0, ZS * 2)],
                        out_hbm.at[pl.ds((r * BS + s * ZS) * 2, ZS * 2)])

        @pl.when(s == 0)
        def _():
            pltpu.sync_copy(acc.at[pl.ds(NS * ZS * 2, 112)],
                            out_hbm.at[pl.ds((r * BS + NS * ZS) * 2, 112)])

        @pl.when((s == 1) & (r < NB - 1))
        def _():
            pltpu.sync_copy(acc.at[pl.ds(2488 * 2, 32)],
                            out_hbm.at[pl.ds((r * BS + 2488) * 2, 32)])

        plsc.subcore_barrier()


_RB = 1000


def _mm_body(x_ref, w_ref, o_ref):
    o_ref[...] = jnp.dot(x_ref[...], w_ref[...],
                         preferred_element_type=jnp.float32)


def _matmul(x, w):
    return pl.pallas_call(
        _mm_body,
        grid=(N // _RB,),
        in_specs=[pl.BlockSpec((_RB, D), lambda i: (i, 0)),
                  pl.BlockSpec((D, D), lambda i: (0, 0))],
        out_specs=pl.BlockSpec((_RB, D), lambda i: (i, 0)),
        out_shape=jax.ShapeDtypeStruct((N, D), jnp.float32),
    )(x, w)


def _scale_body(h_ref, deg_ref, o_ref):
    o_ref[...] = h_ref[...] * lax.rsqrt(deg_ref[...])


def _scale(h, deg):
    return pl.pallas_call(
        _scale_body,
        grid=(N // _RB,),
        in_specs=[pl.BlockSpec((_RB, D), lambda i: (i, 0)),
                  pl.BlockSpec((_RB, 1), lambda i: (i, 0))],
        out_specs=pl.BlockSpec((_RB, D), lambda i: (i, 0)),
        out_shape=jax.ShapeDtypeStruct((N, D), jnp.float32),
    )(h, deg)


def _mid_body(m_ref, g_ref, deg_ref, b_ref, w_ref, o_ref):
    dis = lax.rsqrt(deg_ref[...])
    h = jnp.maximum(dis * (m_ref[...] + g_ref[...]) + b_ref[...], 0.0)
    o_ref[...] = dis * jnp.dot(h, w_ref[...],
                               preferred_element_type=jnp.float32)


def _mid(m1, g1, deg, b1, w2):
    return pl.pallas_call(
        _mid_body,
        grid=(N // _RB,),
        in_specs=[pl.BlockSpec((_RB, D), lambda i: (i, 0)),
                  pl.BlockSpec((_RB, D), lambda i: (i, 0)),
                  pl.BlockSpec((_RB, 1), lambda i: (i, 0)),
                  pl.BlockSpec((1, D), lambda i: (0, 0)),
                  pl.BlockSpec((D, D), lambda i: (0, 0))],
        out_specs=pl.BlockSpec((_RB, D), lambda i: (i, 0)),
        out_shape=jax.ShapeDtypeStruct((N, D), jnp.float32),
    )(m1, g1, deg, b1, w2)


def _final_body(m_ref, g_ref, deg_ref, b_ref, o_ref):
    o_ref[...] = (lax.rsqrt(deg_ref[...]) * (m_ref[...] + g_ref[...])
                  + b_ref[...])


def _final(m2, g2, deg, b2):
    return pl.pallas_call(
        _final_body,
        grid=(N // _RB,),
        in_specs=[pl.BlockSpec((_RB, D), lambda i: (i, 0)),
                  pl.BlockSpec((_RB, D), lambda i: (i, 0)),
                  pl.BlockSpec((_RB, 1), lambda i: (i, 0)),
                  pl.BlockSpec((1, D), lambda i: (0, 0))],
        out_specs=pl.BlockSpec((_RB, D), lambda i: (i, 0)),
        out_shape=jax.ShapeDtypeStruct((N, D), jnp.float32),
    )(m2, g2, deg, b2)


def kernel(x, edge_index, W1, b1, W2, b2):
    src = edge_index[0].astype(jnp.int32)
    dst = edge_index[1].astype(jnp.int32)

    seg_src, seg_dst, counts, hist = _route(src, dst)
    h1 = _matmul(x, W1)
    deg = hist[:, :1] + 1.0
    g1 = _scale(h1, deg)
    m1 = _edge_stage(g1.reshape(N * 2, 128), seg_src, seg_dst, counts)
    g2 = _mid(m1.reshape(N, D), g1, deg, b1.reshape(1, D), W2)
    m2 = _edge_stage(g2.reshape(N * 2, 128), seg_src, seg_dst, counts)
    return _final(m2.reshape(N, D), g2, deg, b2.reshape(1, D))
